# baseline (speedup 1.0000x reference)
import functools

import jax
import jax.numpy as jnp
from jax import lax
from jax.experimental import pallas as pl
from jax.experimental.pallas import tpu as pltpu
from jax.experimental.pallas import tpu_sc as plsc

B, F, D = 16384, 26, 32
CARD = 100000
TOTAL = B * F
NW = 32
RW = TOTAL // NW
CHUNK = 1664
NCHUNK = RW // CHUNK
GPC = CHUNK // 128
PER = 208

@functools.lru_cache(maxsize=1)
def _build():
    mesh = plsc.VectorSubcoreMesh(core_axis_name="c", subcore_axis_name="s")
    return functools.partial(
        pl.kernel,
        out_type=jax.ShapeDtypeStruct((TOTAL, D), jnp.float32),
        mesh=mesh,
        scratch_types=[
            pltpu.VMEM((CHUNK,), jnp.int32),
            pltpu.VMEM((CHUNK, D), jnp.float32),
            pltpu.VMEM((PER,), jnp.int32),
            pltpu.VMEM((F * D,), jnp.float32),
            pltpu.SemaphoreType.DMA,
        ],
        compiler_params=pltpu.CompilerParams(use_tc_tiling_on_sc=False),
    )(_embed_body)


def _embed_body(x_hbm, table_hbm, shared_hbm, out_hbm,
                idx_v, rows_v, offs_v, shared_v, sem):
    wid = lax.axis_index("s") * 2 + lax.axis_index("c")
    base = wid * RW

    pltpu.sync_copy(shared_hbm, shared_v)

    for k in range(PER // 16):
        v = lax.iota(jnp.int32, 16) + (16 * k)
        offs_v[pl.ds(16 * k, 16)] = (v % 26) * CARD

    def chunk_body(c, carry):
        start = base + c * CHUNK

        pltpu.sync_copy(x_hbm.at[pl.ds(start, CHUNK)], idx_v)

        for v in range(CHUNK // 16):
            idx_v[pl.ds(16 * v, 16)] = (
                idx_v[pl.ds(16 * v, 16)]
                + offs_v[pl.ds((16 * v) % PER, 16)])

        descs = [
            pltpu.async_copy(table_hbm.at[idx_v.at[pl.ds(j * 128, 128)]],
                             rows_v.at[pl.ds(j * 128, 128)], sem)
            for j in range(GPC)
        ]
        for d in descs:
            d.wait()

        def group_body(g, carry2):
            row0 = g * 26
            for r in range(26):
                for h in range(2):
                    rows_v[row0 + r, pl.ds(h * 16, 16)] = (
                        rows_v[row0 + r, pl.ds(h * 16, 16)]
                        + shared_v[pl.ds(r * D + h * 16, 16)])
            return carry2

        lax.fori_loop(0, CHUNK // 26, group_body, 0)

        pltpu.sync_copy(rows_v, out_hbm.at[pl.ds(start, CHUNK)])
        return carry

    lax.fori_loop(0, NCHUNK, chunk_body, 0)


def kernel(x, indiv_embed, shared_embed):
    x1 = x.reshape(TOTAL)
    sh = shared_embed.reshape(F * D)
    out = _build()(x1, indiv_embed, sh)
    return out.reshape(B, F, D)
